# baseline (speedup 1.0000x reference)
import functools

import jax
import jax.numpy as jnp
from jax import lax
from jax.experimental import pallas as pl
from jax.experimental.pallas import tpu as pltpu
from jax.experimental.pallas import tpu_sc as plsc

_NC = 2
_NS = 16
_NW = _NC * _NS
_L = 16


def kernel(x, arr, index):
    del x
    t, r, d = arr.shape
    n = r * d
    chunk = n // _NW
    arr2 = arr.reshape(t, n)
    idx16 = jnp.broadcast_to(jnp.asarray(index, jnp.int32), (_L,))

    mesh = plsc.VectorSubcoreMesh(
        core_axis_name="c", subcore_axis_name="s",
        num_cores=_NC, num_subcores=_NS,
    )

    @functools.partial(
        pl.kernel,
        out_type=jax.ShapeDtypeStruct((n,), jnp.float32),
        mesh=mesh,
        scratch_types=[
            pltpu.VMEM((_L,), jnp.int32),
            pltpu.SemaphoreType.DMA,
        ],
    )
    def body(idx_hbm, arr_hbm, out_hbm, idx_v, sem):
        pltpu.sync_copy(idx_hbm, idx_v)
        i = idx_v[...][0]
        wid = lax.axis_index("s") * _NC + lax.axis_index("c")
        base = wid * chunk
        pltpu.async_copy(
            arr_hbm.at[i, pl.ds(base, chunk)],
            out_hbm.at[pl.ds(base, chunk)],
            sem,
        ).wait()

    return body(idx16, arr2).reshape(r, d)


# speedup vs baseline: 1.2202x; 1.2202x over previous
import functools

import jax
import jax.numpy as jnp
from jax import lax
from jax.experimental import pallas as pl
from jax.experimental.pallas import tpu as pltpu
from jax.experimental.pallas import tpu_sc as plsc

_NC = 2
_NS = 16
_NW = _NC * _NS
_L = 16


def kernel(x, arr, index):
    del x
    t, r, d = arr.shape
    rows = r // _NW
    idx16 = jnp.broadcast_to(jnp.asarray(index, jnp.int32), (_L,))

    mesh = plsc.VectorSubcoreMesh(
        core_axis_name="c", subcore_axis_name="s",
        num_cores=_NC, num_subcores=_NS,
    )

    @functools.partial(
        pl.kernel,
        out_type=jax.ShapeDtypeStruct((r, d), jnp.float32),
        mesh=mesh,
        scratch_types=[
            pltpu.VMEM((_L,), jnp.int32),
            pltpu.SemaphoreType.DMA,
        ],
    )
    def body(idx_hbm, arr_hbm, out_hbm, idx_v, sem):
        pltpu.sync_copy(idx_hbm, idx_v)
        i = idx_v[...][0]
        wid = lax.axis_index("s") * _NC + lax.axis_index("c")
        base = wid * rows
        pltpu.async_copy(
            arr_hbm.at[i, pl.ds(base, rows), :],
            out_hbm.at[pl.ds(base, rows), :],
            sem,
        ).wait()

    return body(idx16, arr)


# speedup vs baseline: 1.4651x; 1.2007x over previous
import functools

import jax
import jax.numpy as jnp
from jax import lax
from jax.experimental import pallas as pl
from jax.experimental.pallas import tpu as pltpu
from jax.experimental.pallas import tpu_sc as plsc

_NC = 2
_NS = 16
_NW = _NC * _NS
_L = 16


def kernel(x, arr, index):
    del x
    t, r, d = arr.shape
    rows = r // _NW
    idx16 = jnp.broadcast_to(jnp.asarray(index, jnp.int32), (_L,))

    mesh = plsc.VectorSubcoreMesh(
        core_axis_name="c", subcore_axis_name="s",
        num_cores=_NC, num_subcores=_NS,
    )

    @functools.partial(
        pl.kernel,
        out_type=jax.ShapeDtypeStruct((r, d), jnp.float32),
        mesh=mesh,
        scratch_types=[
            pltpu.VMEM((_L,), jnp.int32),
            pltpu.VMEM((r // _NW, 64), jnp.float32),
            pltpu.SemaphoreType.DMA,
        ],
    )
    def body(idx_hbm, arr_hbm, out_hbm, idx_v, row_v, sem):
        pltpu.sync_copy(idx_hbm, idx_v)
        i = idx_v[...][0]
        wid = lax.axis_index("s") * _NC + lax.axis_index("c")
        base = wid * rows
        pltpu.async_copy(arr_hbm.at[i, pl.ds(base, rows), :], row_v, sem).wait()
        pltpu.async_copy(row_v, out_hbm.at[pl.ds(base, rows), :], sem).wait()

    return body(idx16, arr)
